# baseline (speedup 1.0000x reference)
import functools

import jax
import jax.numpy as jnp
import numpy as np
from jax import lax
from jax.experimental import pallas as pl
from jax.experimental.pallas import tpu as pltpu
from jax.experimental.pallas import tpu_sc as plsc

_NTYPES = 2
_SEL = [16, 32]
_SEC = [0, 16, 48]
_NNEI = 48
_RCUT = 6.0
_RCUT_SMTH = 0.5
_ENV_PROT = 0.01
_NLOC = 2048
_NALL = 4096
_NW = 32
_APT = _NLOC // _NW
_ABLK = 128
_CH = 64

_TRI00 = 136
_TRI01 = 512
_TRI11 = 528
_NROWS = _TRI00 + _TRI01 + _TRI11


def _tri_diag_order(s):
    order = [0, s // 2]
    for d in range(1, s // 2):
        order += [d, s - d]
    return order


def _row_weights():
    def tri_w(s, scale):
        rows = []
        for d in _tri_diag_order(s):
            rows += [(1.0 if d == 0 else 2.0) * scale] * (s - d)
        return rows

    return np.asarray(
        tri_w(16, 1.0 / 256) + [1.0 / 512] * _TRI01 + tri_w(32, 1.0 / 1024),
        dtype=np.float32).reshape(_NROWS, 1)


def _sc_gather_kernel(nlist_hbm, coord_hbm, out_hbm, coord_v, nlist_v, drr_v):
    wid = lax.axis_index("s") * 2 + lax.axis_index("c")
    strip = wid // 2
    half = wid % 2
    base = strip * _ABLK
    pltpu.sync_copy(coord_hbm, coord_v)
    pltpu.sync_copy(nlist_hbm.at[pl.ds(base * _NNEI, _ABLK * _NNEI)], nlist_v)
    iota = lax.iota(jnp.int32, 16)
    zero = jnp.zeros((16,), jnp.int32)
    for v in range(_ABLK // 16):
        gids3 = (iota + (base + v * 16)) * 3
        clx = plsc.load_gather(coord_v, [gids3])
        cly = plsc.load_gather(coord_v, [gids3 + 1])
        clz = plsc.load_gather(coord_v, [gids3 + 2])
        ccol = iota + (v * 16)
        nbase = (iota + (v * 16)) * _NNEI

        @pl.when(half == 0)
        def _(clx=clx, cly=cly, ccol=ccol, nbase=nbase):
            @plsc.parallel_loop(0, _NNEI, unroll=4)
            def body(n):
                nl3 = plsc.load_gather(nlist_v, [nbase + n]) * 3
                crx = plsc.load_gather(coord_v, [nl3])
                cry = plsc.load_gather(coord_v, [nl3 + 1])
                plsc.store_scatter(drr_v, [zero + n, ccol], crx - clx)
                plsc.store_scatter(drr_v, [zero + (_NNEI + n), ccol],
                                   cry - cly)

        @pl.when(half == 1)
        def _(clx=clx, cly=cly, clz=clz, ccol=ccol, nbase=nbase):
            @plsc.parallel_loop(0, _NNEI, unroll=4)
            def body(n):
                nl3 = plsc.load_gather(nlist_v, [nbase + n]) * 3
                crx = plsc.load_gather(coord_v, [nl3])
                cry = plsc.load_gather(coord_v, [nl3 + 1])
                crz = plsc.load_gather(coord_v, [nl3 + 2])
                dx = crx - clx
                dy = cry - cly
                dz = crz - clz
                r2 = dx * dx + dy * dy + dz * dz
                plsc.store_scatter(drr_v, [zero + n, ccol], dz)
                plsc.store_scatter(drr_v, [zero + (_NNEI + n), ccol], r2)

    pltpu.sync_copy(
        drr_v, out_hbm.at[pl.ds(half * 2 * _NNEI, 2 * _NNEI),
                          pl.ds(base, _ABLK)])


def _sc_gather(nlist2, coord):
    mesh = plsc.VectorSubcoreMesh(
        core_axis_name="c", subcore_axis_name="s", num_cores=2, num_subcores=16
    )
    fn = pl.kernel(
        _sc_gather_kernel,
        out_type=jax.ShapeDtypeStruct((4 * _NNEI, _NLOC), jnp.float32),
        mesh=mesh,
        compiler_params=pltpu.CompilerParams(needs_layout_passes=False),
        scratch_types=[
            pltpu.VMEM((_NALL * 3,), jnp.float32),
            pltpu.VMEM((_ABLK * _NNEI,), jnp.int32),
            pltpu.VMEM((2 * _NNEI, _ABLK), jnp.float32),
        ],
    )
    return fn(nlist2.reshape(-1), coord.reshape(-1))


def _tc_body(drr_ref, aty_ref, mean_ref, std_ref, wt_ref, w0_ref, c0_ref,
             w1_ref, c1_ref, w2_ref, c2_ref, res_ref, sw_ref, e_scr):
    A = _ABLK
    drr = drr_ref[...]
    dx = drr[0]
    dy = drr[1]
    dz = drr[2]
    r2 = drr[3]
    r = jnp.sqrt(r2)
    uu = (r - _RCUT_SMTH) / (_RCUT - _RCUT_SMTH)
    vv = uu * uu * uu * (-6.0 * uu * uu + 15.0 * uu - 10.0) + 1.0
    sw = jnp.where(r <= _RCUT_SMTH, 1.0, jnp.where(r >= _RCUT, 0.0, vv))
    sw_ref[...] = sw
    re = r + _ENV_PROT
    fac = sw / (re * re)
    ex = dx * fac
    ey = dy * fac
    ez = dz * fac
    mean_all = mean_ref[...]
    std_all = std_ref[...]
    aty = aty_ref[...].reshape(1, A)
    is0 = aty == 0

    def norm(e, c):
        m0 = mean_all[0, :, c][:, None]
        m1 = mean_all[1, :, c][:, None]
        s0 = std_all[0, :, c][:, None]
        s1 = std_all[1, :, c][:, None]
        return jnp.where(is0, (e - m0) / s0, (e - m1) / s1)

    rx = norm(ex, 1)
    ry = norm(ey, 2)
    rz = norm(ez, 3)

    def gram(i0, i1, j0, j1):
        return (rx[i0:i1, None, :] * rx[None, j0:j1, :]
                + ry[i0:i1, None, :] * ry[None, j0:j1, :]
                + rz[i0:i1, None, :] * rz[None, j0:j1, :]
                ).reshape((i1 - i0) * (j1 - j0), A)

    def diag(base, s, d):
        return (rx[base:base + s - d, :] * rx[base + d:base + s, :]
                + ry[base:base + s - d, :] * ry[base + d:base + s, :]
                + rz[base:base + s - d, :] * rz[base + d:base + s, :])

    def put_tri(base, s, off):
        order = _tri_diag_order(s)
        pieces = [diag(base, s, order[0]), diag(base, s, order[1])]
        for d in order[2:][::2]:
            pieces.append(jnp.concatenate(
                [diag(base, s, d), diag(base, s, s - d)], axis=0))
        for p in pieces:
            e_scr[off:off + p.shape[0], :] = p
            off += p.shape[0]
        return off

    off = put_tri(0, 16, 0)
    e_scr[off:off + _TRI01, :] = gram(0, 16, 16, 48)
    off += _TRI01
    put_tri(16, 32, off)

    def make_body(pr, row0, ch):
        def body(i, acc):
            base = row0 + i * ch
            ec = e_scr[pl.ds(base, ch), :]
            ew = (ec * wt_ref[pl.ds(base, ch), :]).reshape(ch // 8, 8, A)
            l0 = [jnp.tanh(ec * w0_ref[pr, k] + c0_ref[pr, k])
                  for k in range(4)]
            t1 = []
            for m in range(8):
                h = l0[0] * w1_ref[pr, m]
                for k in range(1, 4):
                    h = h + l0[k] * w1_ref[pr, k * 8 + m]
                t1.append(jnp.tanh(h + c1_ref[pr, m]))
            l1 = [t1[m] + l0[m % 4] for m in range(8)]
            t2 = []
            for p in range(16):
                h = l1[0] * w2_ref[pr, p]
                for m in range(1, 8):
                    h = h + l1[m] * w2_ref[pr, m * 16 + p]
                t2.append(jnp.tanh(h + c2_ref[pr, p]))

            def pred(x):
                return jnp.sum(ew * x.reshape(ch // 8, 8, A), axis=0)

            s0 = [pred(l0[k]) for k in range(4)]
            s1 = [pred(t1[m]) for m in range(8)]
            s2 = [pred(t2[p]) for p in range(16)]
            contrib = jnp.concatenate(
                [(s2[p] + s1[p % 8] + s0[p % 4])[None] for p in range(16)],
                axis=0)
            return acc + contrib
        return body

    acc = jnp.zeros((16, 8, A), jnp.float32)
    acc = lax.fori_loop(0, 1, make_body(0, 0, _TRI00), acc)
    acc = lax.fori_loop(0, _TRI01 // 64, make_body(1, _TRI00, 64), acc)
    acc = lax.fori_loop(0, _TRI11 // 88,
                        make_body(3, _TRI00 + _TRI01, 88), acc)
    res_ref[...] = jnp.sum(acc, axis=1)


def _tc_dense(drr, aty2, mean, stddev, w0, c0, w1, c1, w2, c2, interpret=False):
    nblk = _NLOC // _ABLK
    grid = (nblk,)
    smem = pl.BlockSpec(memory_space=pltpu.SMEM)
    wt = jnp.asarray(_row_weights())
    return pl.pallas_call(
        _tc_body,
        grid=grid,
        in_specs=[
            pl.BlockSpec((4, _NNEI, _ABLK), lambda i: (0, 0, i)),
            pl.BlockSpec((1, 1, _ABLK), lambda i: (i, 0, 0)),
            pl.BlockSpec((2, _NNEI, 4), lambda i: (0, 0, 0)),
            pl.BlockSpec((2, _NNEI, 4), lambda i: (0, 0, 0)),
            pl.BlockSpec((_NROWS, 1), lambda i: (0, 0)),
            smem, smem, smem, smem, smem, smem,
        ],
        out_specs=[
            pl.BlockSpec((16, _ABLK), lambda i: (0, i)),
            pl.BlockSpec((_NNEI, _ABLK), lambda i: (0, i)),
        ],
        out_shape=[
            jax.ShapeDtypeStruct((16, _NLOC), jnp.float32),
            jax.ShapeDtypeStruct((_NNEI, _NLOC), jnp.float32),
        ],
        scratch_shapes=[pltpu.VMEM((_NROWS, _ABLK), jnp.float32)],
        interpret=interpret,
    )(drr.reshape(4, _NNEI, _NLOC), aty2, mean, stddev, wt,
      w0, c0, w1, c1, w2, c2)


def kernel(nlist, extended_coord, extended_atype, mean, stddev,
           W0, b0, W1, b1, W2, b2):
    nb, nloc, nnei = nlist.shape
    nlist2 = nlist.reshape(nloc, nnei).astype(jnp.int32)
    coord = extended_coord.reshape(-1, 3)
    drr = _sc_gather(nlist2, coord)
    aty2 = extended_atype[0, :nloc].astype(jnp.int32).reshape(
        _NLOC // _ABLK, 1, _ABLK)
    w0 = W0.reshape(4, 4)
    c0 = b0.reshape(4, 4)
    w1 = W1.reshape(4, 32)
    c1 = b1.reshape(4, 8)
    w2 = W2.reshape(4, 128)
    c2 = b2.reshape(4, 16)
    res_t, sw_t = _tc_dense(drr, aty2, mean, stddev, w0, c0, w1, c1, w2, c2)
    result = res_t.T.reshape(nb, nloc, 16)
    sw = sw_t.T.reshape(nb, nloc, nnei)
    return result, sw


# speedup vs baseline: 1.0039x; 1.0039x over previous
import functools

import jax
import jax.numpy as jnp
import numpy as np
from jax import lax
from jax.experimental import pallas as pl
from jax.experimental.pallas import tpu as pltpu
from jax.experimental.pallas import tpu_sc as plsc

_NTYPES = 2
_SEL = [16, 32]
_SEC = [0, 16, 48]
_NNEI = 48
_RCUT = 6.0
_RCUT_SMTH = 0.5
_ENV_PROT = 0.01
_NLOC = 2048
_NALL = 4096
_NW = 32
_APT = _NLOC // _NW
_ABLK = 128
_CH = 64

_TRI00 = 136
_TRI01 = 512
_TRI11 = 528
_NROWS = _TRI00 + _TRI01 + _TRI11


def _tri_diag_order(s):
    order = [0, s // 2]
    for d in range(1, s // 2):
        order += [d, s - d]
    return order


def _row_weights():
    def tri_w(s, scale):
        rows = []
        for d in _tri_diag_order(s):
            rows += [(1.0 if d == 0 else 2.0) * scale] * (s - d)
        return rows

    return np.asarray(
        tri_w(16, 1.0 / 256) + [1.0 / 512] * _TRI01 + tri_w(32, 1.0 / 1024),
        dtype=np.float32).reshape(_NROWS, 1)


def _sc_gather_kernel(nlist_hbm, coord_hbm, out_hbm, coord_v, nlist_v, drr_v):
    wid = lax.axis_index("s") * 2 + lax.axis_index("c")
    strip = wid // 2
    half = wid % 2
    base = strip * _ABLK
    pltpu.sync_copy(coord_hbm, coord_v)
    pltpu.sync_copy(nlist_hbm.at[pl.ds(base * _NNEI, _ABLK * _NNEI)], nlist_v)
    iota = lax.iota(jnp.int32, 16)
    zero = jnp.zeros((16,), jnp.int32)
    for v in range(_ABLK // 16):
        gids3 = (iota + (base + v * 16)) * 3
        clx = plsc.load_gather(coord_v, [gids3])
        cly = plsc.load_gather(coord_v, [gids3 + 1])
        clz = plsc.load_gather(coord_v, [gids3 + 2])
        ccol = iota + (v * 16)
        nbase = (iota + (v * 16)) * _NNEI

        @pl.when(half == 0)
        def _(clx=clx, cly=cly, ccol=ccol, nbase=nbase):
            @plsc.parallel_loop(0, _NNEI, unroll=2)
            def body(n):
                nl3 = plsc.load_gather(nlist_v, [nbase + n]) * 3
                crx = plsc.load_gather(coord_v, [nl3])
                cry = plsc.load_gather(coord_v, [nl3 + 1])
                plsc.store_scatter(drr_v, [zero + n, ccol], crx - clx)
                plsc.store_scatter(drr_v, [zero + (_NNEI + n), ccol],
                                   cry - cly)

        @pl.when(half == 1)
        def _(clx=clx, cly=cly, clz=clz, ccol=ccol, nbase=nbase):
            @plsc.parallel_loop(0, _NNEI, unroll=2)
            def body(n):
                nl3 = plsc.load_gather(nlist_v, [nbase + n]) * 3
                crx = plsc.load_gather(coord_v, [nl3])
                cry = plsc.load_gather(coord_v, [nl3 + 1])
                crz = plsc.load_gather(coord_v, [nl3 + 2])
                dx = crx - clx
                dy = cry - cly
                dz = crz - clz
                r2 = dx * dx + dy * dy + dz * dz
                plsc.store_scatter(drr_v, [zero + n, ccol], dz)
                plsc.store_scatter(drr_v, [zero + (_NNEI + n), ccol], r2)

    pltpu.sync_copy(
        drr_v, out_hbm.at[pl.ds(half * 2 * _NNEI, 2 * _NNEI),
                          pl.ds(base, _ABLK)])


def _sc_gather(nlist2, coord):
    mesh = plsc.VectorSubcoreMesh(
        core_axis_name="c", subcore_axis_name="s", num_cores=2, num_subcores=16
    )
    fn = pl.kernel(
        _sc_gather_kernel,
        out_type=jax.ShapeDtypeStruct((4 * _NNEI, _NLOC), jnp.float32),
        mesh=mesh,
        compiler_params=pltpu.CompilerParams(needs_layout_passes=False),
        scratch_types=[
            pltpu.VMEM((_NALL * 3,), jnp.float32),
            pltpu.VMEM((_ABLK * _NNEI,), jnp.int32),
            pltpu.VMEM((2 * _NNEI, _ABLK), jnp.float32),
        ],
    )
    return fn(nlist2.reshape(-1), coord.reshape(-1))


def _tc_body(drr_ref, aty_ref, mean_ref, std_ref, wt_ref, w0_ref, c0_ref,
             w1_ref, c1_ref, w2_ref, c2_ref, res_ref, sw_ref, e_scr):
    A = _ABLK
    drr = drr_ref[...]
    dx = drr[0]
    dy = drr[1]
    dz = drr[2]
    r2 = drr[3]
    r = jnp.sqrt(r2)
    uu = (r - _RCUT_SMTH) / (_RCUT - _RCUT_SMTH)
    vv = uu * uu * uu * (-6.0 * uu * uu + 15.0 * uu - 10.0) + 1.0
    sw = jnp.where(r <= _RCUT_SMTH, 1.0, jnp.where(r >= _RCUT, 0.0, vv))
    sw_ref[...] = sw
    re = r + _ENV_PROT
    fac = sw / (re * re)
    ex = dx * fac
    ey = dy * fac
    ez = dz * fac
    mean_all = mean_ref[...]
    std_all = std_ref[...]
    aty = aty_ref[...].reshape(1, A)
    is0 = aty == 0

    def norm(e, c):
        m0 = mean_all[0, :, c][:, None]
        m1 = mean_all[1, :, c][:, None]
        s0 = std_all[0, :, c][:, None]
        s1 = std_all[1, :, c][:, None]
        return jnp.where(is0, (e - m0) / s0, (e - m1) / s1)

    rx = norm(ex, 1)
    ry = norm(ey, 2)
    rz = norm(ez, 3)

    def gram(i0, i1, j0, j1):
        return (rx[i0:i1, None, :] * rx[None, j0:j1, :]
                + ry[i0:i1, None, :] * ry[None, j0:j1, :]
                + rz[i0:i1, None, :] * rz[None, j0:j1, :]
                ).reshape((i1 - i0) * (j1 - j0), A)

    def diag(base, s, d):
        return (rx[base:base + s - d, :] * rx[base + d:base + s, :]
                + ry[base:base + s - d, :] * ry[base + d:base + s, :]
                + rz[base:base + s - d, :] * rz[base + d:base + s, :])

    def put_tri(base, s, off):
        order = _tri_diag_order(s)
        pieces = [diag(base, s, order[0]), diag(base, s, order[1])]
        for d in order[2:][::2]:
            pieces.append(jnp.concatenate(
                [diag(base, s, d), diag(base, s, s - d)], axis=0))
        for p in pieces:
            e_scr[off:off + p.shape[0], :] = p
            off += p.shape[0]
        return off

    off = put_tri(0, 16, 0)
    e_scr[off:off + _TRI01, :] = gram(0, 16, 16, 48)
    off += _TRI01
    put_tri(16, 32, off)

    def make_body(pr, row0, ch):
        def body(i, acc):
            base = row0 + i * ch
            ec = e_scr[pl.ds(base, ch), :]
            ew = (ec * wt_ref[pl.ds(base, ch), :]).reshape(ch // 8, 8, A)
            l0 = [jnp.tanh(ec * w0_ref[pr, k] + c0_ref[pr, k])
                  for k in range(4)]
            t1 = []
            for m in range(8):
                h = l0[0] * w1_ref[pr, m]
                for k in range(1, 4):
                    h = h + l0[k] * w1_ref[pr, k * 8 + m]
                t1.append(jnp.tanh(h + c1_ref[pr, m]))
            l1 = [t1[m] + l0[m % 4] for m in range(8)]
            t2 = []
            for p in range(16):
                h = l1[0] * w2_ref[pr, p]
                for m in range(1, 8):
                    h = h + l1[m] * w2_ref[pr, m * 16 + p]
                t2.append(jnp.tanh(h + c2_ref[pr, p]))

            def pred(x):
                return jnp.sum(ew * x.reshape(ch // 8, 8, A), axis=0)

            s0 = [pred(l0[k]) for k in range(4)]
            s1 = [pred(t1[m]) for m in range(8)]
            s2 = [pred(t2[p]) for p in range(16)]
            contrib = jnp.concatenate(
                [(s2[p] + s1[p % 8] + s0[p % 4])[None] for p in range(16)],
                axis=0)
            return acc + contrib
        return body

    acc = jnp.zeros((16, 8, A), jnp.float32)
    acc = lax.fori_loop(0, 1, make_body(0, 0, _TRI00), acc)
    acc = lax.fori_loop(0, _TRI01 // 64, make_body(1, _TRI00, 64), acc)
    acc = lax.fori_loop(0, _TRI11 // 88,
                        make_body(3, _TRI00 + _TRI01, 88), acc)
    res_ref[...] = jnp.sum(acc, axis=1)


def _tc_dense(drr, aty2, mean, stddev, w0, c0, w1, c1, w2, c2, interpret=False):
    nblk = _NLOC // _ABLK
    grid = (nblk,)
    smem = pl.BlockSpec(memory_space=pltpu.SMEM)
    wt = jnp.asarray(_row_weights())
    return pl.pallas_call(
        _tc_body,
        grid=grid,
        in_specs=[
            pl.BlockSpec((4, _NNEI, _ABLK), lambda i: (0, 0, i)),
            pl.BlockSpec((1, 1, _ABLK), lambda i: (i, 0, 0)),
            pl.BlockSpec((2, _NNEI, 4), lambda i: (0, 0, 0)),
            pl.BlockSpec((2, _NNEI, 4), lambda i: (0, 0, 0)),
            pl.BlockSpec((_NROWS, 1), lambda i: (0, 0)),
            smem, smem, smem, smem, smem, smem,
        ],
        out_specs=[
            pl.BlockSpec((16, _ABLK), lambda i: (0, i)),
            pl.BlockSpec((_NNEI, _ABLK), lambda i: (0, i)),
        ],
        out_shape=[
            jax.ShapeDtypeStruct((16, _NLOC), jnp.float32),
            jax.ShapeDtypeStruct((_NNEI, _NLOC), jnp.float32),
        ],
        scratch_shapes=[pltpu.VMEM((_NROWS, _ABLK), jnp.float32)],
        interpret=interpret,
    )(drr.reshape(4, _NNEI, _NLOC), aty2, mean, stddev, wt,
      w0, c0, w1, c1, w2, c2)


def kernel(nlist, extended_coord, extended_atype, mean, stddev,
           W0, b0, W1, b1, W2, b2):
    nb, nloc, nnei = nlist.shape
    nlist2 = nlist.reshape(nloc, nnei).astype(jnp.int32)
    coord = extended_coord.reshape(-1, 3)
    drr = _sc_gather(nlist2, coord)
    aty2 = extended_atype[0, :nloc].astype(jnp.int32).reshape(
        _NLOC // _ABLK, 1, _ABLK)
    w0 = W0.reshape(4, 4)
    c0 = b0.reshape(4, 4)
    w1 = W1.reshape(4, 32)
    c1 = b1.reshape(4, 8)
    w2 = W2.reshape(4, 128)
    c2 = b2.reshape(4, 16)
    res_t, sw_t = _tc_dense(drr, aty2, mean, stddev, w0, c0, w1, c1, w2, c2)
    result = res_t.T.reshape(nb, nloc, 16)
    sw = sw_t.T.reshape(nb, nloc, nnei)
    return result, sw
